# baseline (speedup 1.0000x reference)
import functools

import jax
import jax.numpy as jnp
from jax import lax
from jax.experimental import pallas as pl
from jax.experimental.pallas import tpu as pltpu
from jax.experimental.pallas import tpu_sc as plsc

G = 100000
B = 16384
d0, d1, d2 = 4, 8, 16
D0, D1, D2 = d0 * d0, d1 * d1, d2 * d2
OUT_D = D0 + D1 + D2

_info = plsc.get_sparse_core_info()
NC, NS = _info.num_cores, _info.num_subcores
NW = NC * NS
BPW = B // NW
CH = 128
NCH = BPW // CH

_mesh = plsc.VectorSubcoreMesh(core_axis_name="c", subcore_axis_name="s")


@functools.partial(
    pl.kernel,
    mesh=_mesh,
    out_type=(
        jax.ShapeDtypeStruct((B, 2, 8), jnp.float32),
        jax.ShapeDtypeStruct((B, d1, d1), jnp.float32),
        jax.ShapeDtypeStruct((B, d2, d2), jnp.float32),
    ),
    compiler_params=pltpu.CompilerParams(use_tc_tiling_on_sc=False),
    scratch_types=[
        pltpu.VMEM((NCH, CH), jnp.int32),
        pltpu.VMEM((BPW, 2, 8), jnp.float32),
        pltpu.VMEM((BPW, d1, d1), jnp.float32),
        pltpu.VMEM((BPW // 2, d2, d2), jnp.float32),
        pltpu.SemaphoreType.DMA,
        pltpu.SemaphoreType.DMA,
        pltpu.SemaphoreType.DMA,
    ],
)
def _sc_gather(x_hbm, rep0_hbm, rep1_hbm, rep2_hbm,
               out0_hbm, out1_hbm, out2_hbm,
               idx_v, rows0_v, rows1_v, rows2_v, sem0, sem1, sem2):
    wid = lax.axis_index("s") * NC + lax.axis_index("c")
    base = wid * BPW

    pltpu.sync_copy(x_hbm.at[pl.ds(wid * NCH, NCH)], idx_v)

    h2 = [
        pltpu.async_copy(rep2_hbm.at[idx_v.at[j]],
                         rows2_v.at[pl.ds(j * CH, CH)], sem2)
        for j in range(NCH // 2)
    ]
    h0 = [
        pltpu.async_copy(rep0_hbm.at[idx_v.at[j]],
                         rows0_v.at[pl.ds(j * CH, CH)], sem0)
        for j in range(NCH)
    ]
    h1 = [
        pltpu.async_copy(rep1_hbm.at[idx_v.at[j]],
                         rows1_v.at[pl.ds(j * CH, CH)], sem1)
        for j in range(NCH)
    ]

    for h in h2:
        h.wait()
    pltpu.sync_copy(rows2_v, out2_hbm.at[pl.ds(base, BPW // 2)])

    h2b = [
        pltpu.async_copy(rep2_hbm.at[idx_v.at[j]],
                         rows2_v.at[pl.ds((j - NCH // 2) * CH, CH)], sem2)
        for j in range(NCH // 2, NCH)
    ]

    for h in h0:
        h.wait()
    pltpu.sync_copy(rows0_v, out0_hbm.at[pl.ds(base, BPW)])
    for h in h1:
        h.wait()
    pltpu.sync_copy(rows1_v, out1_hbm.at[pl.ds(base, BPW)])

    for h in h2b:
        h.wait()
    pltpu.sync_copy(rows2_v, out2_hbm.at[pl.ds(base + BPW // 2, BPW // 2)])


def kernel(x, rep0, rep1, rep2):
    x2 = x.astype(jnp.int32).reshape(B // CH, CH)
    g0, g1, g2 = _sc_gather(x2, rep0.reshape(G, 2, 8), rep1, rep2)
    return jnp.concatenate(
        [g0.reshape(B, D0), g1.reshape(B, D1), g2.reshape(B, D2)], axis=1)


# speedup vs baseline: 4.9882x; 4.9882x over previous
import functools

import jax
import jax.numpy as jnp
from jax import lax
from jax.experimental import pallas as pl
from jax.experimental.pallas import tpu as pltpu
from jax.experimental.pallas import tpu_sc as plsc

G = 100000
B = 16384
D0, D1, D2 = 16, 64, 256
OUT_D = D0 + D1 + D2
DP = 128

_info = plsc.get_sparse_core_info()
NC, NS = _info.num_cores, _info.num_subcores
NW = NC * NS
BPW = B // NW
CH = 128
NCH = BPW // CH

_mesh = plsc.VectorSubcoreMesh(core_axis_name="c", subcore_axis_name="s")


@functools.partial(
    pl.kernel,
    mesh=_mesh,
    out_type=(
        jax.ShapeDtypeStruct((B, DP), jnp.float32),
        jax.ShapeDtypeStruct((B, DP), jnp.float32),
        jax.ShapeDtypeStruct((B, D2), jnp.float32),
    ),
    compiler_params=pltpu.CompilerParams(use_tc_tiling_on_sc=True),
    scratch_types=[
        pltpu.VMEM((NCH, CH), jnp.int32),
        pltpu.VMEM((CH, DP), jnp.float32),
        pltpu.VMEM((CH, DP), jnp.float32),
        pltpu.VMEM((2 * CH, D2), jnp.float32),
        pltpu.SemaphoreType.DMA,
        pltpu.SemaphoreType.DMA,
        pltpu.SemaphoreType.DMA,
    ],
)
def _sc_gather(x_hbm, rep0_hbm, rep1_hbm, rep2_hbm,
               out0_hbm, out1_hbm, out2_hbm,
               idx_v, rows0_v, rows1_v, rows2_v, sem0, sem1, sem2):
    wid = lax.axis_index("s") * NC + lax.axis_index("c")
    base = wid * BPW

    pltpu.sync_copy(x_hbm.at[pl.ds(wid * NCH, NCH)], idx_v)

    def fire2(j):
        return pltpu.async_copy(rep2_hbm.at[idx_v.at[j]],
                                rows2_v.at[pl.ds((j % 2) * CH, CH)], sem2)

    def fire0(j):
        return pltpu.async_copy(rep0_hbm.at[idx_v.at[j]], rows0_v, sem0)

    def fire1(j):
        return pltpu.async_copy(rep1_hbm.at[idx_v.at[j]], rows1_v, sem1)

    h2 = fire2(0)
    h0 = fire0(0)
    h1 = fire1(0)
    h2n = fire2(1)
    for j in range(NCH):
        h2.wait()
        pltpu.sync_copy(rows2_v.at[pl.ds((j % 2) * CH, CH)],
                        out2_hbm.at[pl.ds(base + j * CH, CH)])
        h2 = h2n
        if j + 2 < NCH:
            h2n = fire2(j + 2)
        h0.wait()
        pltpu.sync_copy(rows0_v, out0_hbm.at[pl.ds(base + j * CH, CH)])
        if j + 1 < NCH:
            h0 = fire0(j + 1)
        h1.wait()
        pltpu.sync_copy(rows1_v, out1_hbm.at[pl.ds(base + j * CH, CH)])
        if j + 1 < NCH:
            h1 = fire1(j + 1)


def kernel(x, rep0, rep1, rep2):
    x2 = x.astype(jnp.int32).reshape(B // CH, CH)
    r0 = jnp.pad(rep0.reshape(G, D0), ((0, 0), (0, DP - D0)))
    r1 = jnp.pad(rep1.reshape(G, D1), ((0, 0), (0, DP - D1)))
    g0, g1, g2 = _sc_gather(x2, r0, r1, rep2.reshape(G, D2))
    return jnp.concatenate([g0[:, :D0], g1[:, :D1], g2], axis=1)
